# baseline (speedup 1.0000x reference)
import functools

import jax
import jax.numpy as jnp
import numpy as np
from jax import lax
from jax.experimental import pallas as pl
from jax.experimental.pallas import tpu as pltpu
from jax.experimental.pallas import tpu_sc as plsc

_N = 8192
_D = 128
_CAPACITY = 2048


def _compute_src() -> np.ndarray:
    def step(carry, t):
        buf, size, key = carry
        key, kc, ki = jax.random.split(key, 3)

        def not_full(_):
            return buf.at[size].set(t), t

        def full(_):
            coin = jax.random.uniform(kc) < 0.5

            def swap(_):
                idx = jax.random.randint(ki, (), 0, _CAPACITY)
                selected = buf[idx]
                ar = jnp.arange(_CAPACITY)
                gidx = jnp.clip(jnp.where(ar < idx, ar, ar + 1), 0, _CAPACITY - 1)
                return buf[gidx].at[_CAPACITY - 1].set(t), selected

            def keep(_):
                return buf, t

            return lax.cond(coin, swap, keep, None)

        new_buf, out = lax.cond(size < _CAPACITY, not_full, full, None)
        return (new_buf, jnp.minimum(size + 1, _CAPACITY), key), out

    def run():
        buf0 = jnp.zeros((_CAPACITY,), dtype=jnp.int32)
        ts = jnp.arange(_N, dtype=jnp.int32)
        (_, _, _), src = lax.scan(step, (buf0, jnp.int32(0), jax.random.key(42)), ts)
        return src

    return np.asarray(jax.jit(run)())


_SRC = _compute_src()

_INFO = plsc.get_sparse_core_info()
_NC, _NS = _INFO.num_cores, _INFO.num_subcores
_NW = _NC * _NS
_GN = _N - _CAPACITY
_GROWS_W = _GN // _NW
_IROWS_W = _CAPACITY // _NW


@functools.partial(
    pl.kernel,
    out_type=jax.ShapeDtypeStruct((_N, _D), jnp.float32),
    mesh=plsc.VectorSubcoreMesh(core_axis_name="c", subcore_axis_name="s"),
    scratch_types=[
        pltpu.VMEM((_GROWS_W,), jnp.int32),
        pltpu.VMEM((_GROWS_W, _D), jnp.float32),
        pltpu.SemaphoreType.DMA,
        pltpu.SemaphoreType.DMA,
    ],
)
def _history_gather(z_hbm, src_hbm, out_hbm, idx_v, rows_v, sem, sem_d):
    wid = lax.axis_index("s") * _NC + lax.axis_index("c")
    ibase = wid * _IROWS_W
    dcp = pltpu.async_copy(z_hbm.at[pl.ds(ibase, _IROWS_W)],
                           out_hbm.at[pl.ds(ibase, _IROWS_W)], sem_d)
    gbase = wid * _GROWS_W
    pltpu.sync_copy(src_hbm.at[pl.ds(gbase, _GROWS_W)], idx_v)
    pltpu.async_copy(z_hbm.at[idx_v], rows_v, sem).wait()
    pltpu.sync_copy(rows_v, out_hbm.at[pl.ds(_CAPACITY + gbase, _GROWS_W)])
    dcp.wait()


def kernel(z_prime):
    src_tail = jnp.asarray(_SRC[_CAPACITY:])
    return _history_gather(z_prime, src_tail)


# speedup vs baseline: 2.2155x; 2.2155x over previous
import functools

import jax
import jax.numpy as jnp
import numpy as np
from jax import lax
from jax.experimental import pallas as pl
from jax.experimental.pallas import tpu as pltpu
from jax.experimental.pallas import tpu_sc as plsc

_N = 8192
_D = 128
_CAPACITY = 2048


def _compute_src() -> np.ndarray:
    def step(carry, t):
        buf, size, key = carry
        key, kc, ki = jax.random.split(key, 3)

        def not_full(_):
            return buf.at[size].set(t), t

        def full(_):
            coin = jax.random.uniform(kc) < 0.5

            def swap(_):
                idx = jax.random.randint(ki, (), 0, _CAPACITY)
                selected = buf[idx]
                ar = jnp.arange(_CAPACITY)
                gidx = jnp.clip(jnp.where(ar < idx, ar, ar + 1), 0, _CAPACITY - 1)
                return buf[gidx].at[_CAPACITY - 1].set(t), selected

            def keep(_):
                return buf, t

            return lax.cond(coin, swap, keep, None)

        new_buf, out = lax.cond(size < _CAPACITY, not_full, full, None)
        return (new_buf, jnp.minimum(size + 1, _CAPACITY), key), out

    def run():
        buf0 = jnp.zeros((_CAPACITY,), dtype=jnp.int32)
        ts = jnp.arange(_N, dtype=jnp.int32)
        (_, _, _), src = lax.scan(step, (buf0, jnp.int32(0), jax.random.key(42)), ts)
        return src

    return np.asarray(jax.jit(run)())


_SRC = _compute_src()

_INFO = plsc.get_sparse_core_info()
_NC, _NS = _INFO.num_cores, _INFO.num_subcores
_NW = _NC * _NS
_ROWS_W = _N // _NW


@functools.partial(
    pl.kernel,
    out_type=jax.ShapeDtypeStruct((_N, _D), jnp.float32),
    mesh=plsc.VectorSubcoreMesh(core_axis_name="c", subcore_axis_name="s"),
    compiler_params=pltpu.CompilerParams(use_tc_tiling_on_sc=True),
    scratch_types=[
        pltpu.VMEM((_ROWS_W,), jnp.int32),
        pltpu.VMEM((_ROWS_W, _D), jnp.float32),
        pltpu.SemaphoreType.DMA,
    ],
)
def _history_gather(z_hbm, src_hbm, out_hbm, idx_v, rows_v, sem):
    wid = lax.axis_index("s") * _NC + lax.axis_index("c")
    base = wid * _ROWS_W
    pltpu.sync_copy(src_hbm.at[pl.ds(base, _ROWS_W)], idx_v)
    pltpu.async_copy(z_hbm.at[idx_v], rows_v, sem).wait()
    pltpu.sync_copy(rows_v, out_hbm.at[pl.ds(base, _ROWS_W)])


def kernel(z_prime):
    src = jnp.asarray(_SRC)
    return _history_gather(z_prime, src)
